# baseline (speedup 1.0000x reference)
import functools

import numpy as np
import jax
import jax.numpy as jnp
from jax import lax
from jax.experimental import pallas as pl
from jax.experimental.pallas import tpu as pltpu
from jax.experimental.pallas import tpu_sc as plsc

C = 128
N = 10000
K = 32

NC = 2
NS = 16
NW = NC * NS
L = 16
CV = C // L

TN = 8
NCHUNK = N // TN
BASE_CNT = NCHUNK // NW
EXTRA = NCHUNK - BASE_CNT * NW
MAXCNT = BASE_CNT + 1
IDXPAD = MAXCNT * TN * K


def _prep_body(f_ref, w_ref, b_ref, o_ref):
    acc = lax.dot(f_ref[...], w_ref[...],
                  precision=lax.Precision.HIGHEST,
                  preferred_element_type=jnp.float32)
    h = jnp.maximum(acc + b_ref[...], 0.0)
    bits = lax.bitcast_convert_type(h, jnp.int32)
    rne = (bits + 0x7FFF + ((bits >> 16) & 1)) >> 16
    parts = []
    for g in range(4):
        lo = rne[:, 32 * g:32 * g + 16] & 0xFFFF
        hi = rne[:, 32 * g + 16:32 * g + 32] << 16
        parts.append(lo | hi)
    o_ref[...] = jnp.concatenate(parts, axis=1)


def _tail_body(g_ref, f_ref, w1_ref, b1_ref, w2_ref, b2_ref, o_ref):
    t = lax.dot(g_ref[...], w1_ref[...],
                precision=lax.Precision.HIGHEST,
                preferred_element_type=jnp.float32)
    t = jnp.maximum(t + b1_ref[...], 0.0)
    o = lax.dot(t, w2_ref[...],
                precision=lax.Precision.HIGHEST,
                preferred_element_type=jnp.float32)
    o_ref[...] = jnp.maximum(o + b2_ref[...] + f_ref[...], 0.0)


NBUF = 2
NOBUF = 2


def _sc_core_body(h_hbm, idx_hbm, dlp_hbm, out_hbm,
                  idx_v, dlp_v, rows_v, out_v,
                  dlp_sem, gat_sem, out_sem):
    wid = lax.axis_index("s") * NC + lax.axis_index("c")
    cnt = BASE_CNT + jnp.where(wid < EXTRA, 1, 0)
    start = wid * BASE_CNT + jnp.minimum(wid, EXTRA)

    pltpu.sync_copy(idx_hbm.at[pl.ds(start * TN * K, IDXPAD)], idx_v)

    def dlp_copy(slot, i):
        return pltpu.make_async_copy(
            dlp_hbm.at[pl.ds((start + i) * TN * K * C, TN * K * C)],
            dlp_v.at[slot], dlp_sem.at[slot])

    def gat_copy(slot, i, half):
        off = pl.multiple_of((2 * i + half) * (TN * K // 2), 8)
        return pltpu.make_async_copy(
            h_hbm.at[idx_v.at[pl.ds(off, TN * K // 2)]],
            rows_v.at[slot, pl.ds(half * TN * K // 2, TN * K // 2)],
            gat_sem.at[slot])

    def out_copy(slot, i):
        return pltpu.make_async_copy(
            out_v.at[slot],
            out_hbm.at[pl.ds((start + i) * TN * C, TN * C)], out_sem.at[slot])

    for pre in range(NBUF - 1):
        @pl.when(pre < cnt)
        def _():
            dlp_copy(pre, pre).start()
            gat_copy(pre, pre, 0).start()
            gat_copy(pre, pre, 1).start()

    def chunk_body(i, _):
        s = lax.rem(i, NBUF)
        so = lax.rem(i, NOBUF)

        @pl.when(i + NBUF - 1 < cnt)
        def _():
            dlp_copy(lax.rem(i + NBUF - 1, NBUF), i + NBUF - 1).start()
            gat_copy(lax.rem(i + NBUF - 1, NBUF), i + NBUF - 1, 0).start()
            gat_copy(lax.rem(i + NBUF - 1, NBUF), i + NBUF - 1, 1).start()

        dlp_copy(s, i).wait()
        gat_copy(s, i, 0).wait()
        gat_copy(s, i, 1).wait()

        @pl.when(i >= NOBUF)
        def _():
            out_copy(so, i - NOBUF).wait()

        for nn in range(TN):
            base = nn * K * C

            def row_terms(k):
                terms = []
                for g in range(CV // 2):
                    vi = rows_v[s, nn * K + k, pl.ds(g * L, L)]
                    vb = plsc.bitcast(vi, jnp.bfloat16)
                    a, b = plsc.unpack(vb, format=plsc.PackFormat.INTERLEAVED)
                    terms += [a, b]
                return terms

            def k_body(k, acc):
                off = base + k * C
                t = row_terms(k)
                return tuple(
                    jnp.maximum(acc[j],
                                dlp_v[s, pl.ds(off + j * L, L)] + t[j])
                    for j in range(CV))

            t0 = row_terms(0)
            acc0 = tuple(
                dlp_v[s, pl.ds(base + j * L, L)] + t0[j]
                for j in range(CV))
            acc = lax.fori_loop(1, K, k_body, acc0, unroll=4)
            for j in range(CV):
                out_v[so, pl.ds(nn * C + j * L, L)] = acc[j]

        out_copy(so, i).start()
        return 0

    lax.fori_loop(0, cnt, chunk_body, 0)

    for tail in range(NOBUF):
        @pl.when(cnt > tail)
        def _():
            i = cnt - 1 - tail
            out_copy(lax.rem(i, NOBUF), i).wait()


@functools.cache
def _sc_core():
    return pl.kernel(
        _sc_core_body,
        out_type=jax.ShapeDtypeStruct((N * C,), jnp.float32),
        mesh=plsc.VectorSubcoreMesh(core_axis_name="c", subcore_axis_name="s",
                                    num_cores=NC, num_subcores=NS),
        compiler_params=pltpu.CompilerParams(needs_layout_passes=False,
                                             use_tc_tiling_on_sc=False),
        scratch_types=[
            pltpu.VMEM((IDXPAD,), jnp.int32),
            pltpu.VMEM((NBUF, TN * K * C), jnp.float32),
            pltpu.VMEM((NBUF, TN * K, C // 2), jnp.int32),
            pltpu.VMEM((NOBUF, TN * C), jnp.float32),
            pltpu.SemaphoreType.DMA((NBUF,)),
            pltpu.SemaphoreType.DMA((NBUF,)),
            pltpu.SemaphoreType.DMA((NOBUF,)),
        ],
    )


def kernel(p, f, dlp, idx, W1, b1, Wp1, bp1, Wp2, bp2):
    del p
    fT = f.reshape(C, N).T
    dlp_flat = jnp.transpose(dlp.reshape(C, N, K), (1, 2, 0)).reshape(-1)
    idx_flat = idx.reshape(N * K).astype(jnp.int32)
    idx_flat = jnp.pad(idx_flat, (0, NW * MAXCNT * TN * K - N * K))

    TB = 2000
    grid = (N // TB,)
    hT = pl.pallas_call(
        _prep_body,
        grid=grid,
        in_specs=[
            pl.BlockSpec((TB, C), lambda i: (i, 0)),
            pl.BlockSpec((C, C), lambda i: (0, 0)),
            pl.BlockSpec((1, C), lambda i: (0, 0)),
        ],
        out_specs=pl.BlockSpec((TB, C // 2), lambda i: (i, 0)),
        out_shape=jax.ShapeDtypeStruct((N, C // 2), jnp.int32),
    )(fT, W1.T, b1.reshape(1, C))

    g0 = _sc_core()(hT, idx_flat, dlp_flat).reshape(N, C)

    outT = pl.pallas_call(
        _tail_body,
        grid=grid,
        in_specs=[
            pl.BlockSpec((TB, C), lambda i: (i, 0)),
            pl.BlockSpec((TB, C), lambda i: (i, 0)),
            pl.BlockSpec((C, C), lambda i: (0, 0)),
            pl.BlockSpec((1, C), lambda i: (0, 0)),
            pl.BlockSpec((C, C), lambda i: (0, 0)),
            pl.BlockSpec((1, C), lambda i: (0, 0)),
        ],
        out_specs=pl.BlockSpec((TB, C), lambda i: (i, 0)),
        out_shape=jax.ShapeDtypeStruct((N, C), jnp.float32),
    )(g0, fT, Wp1.T, bp1.reshape(1, C), Wp2.T, bp2.reshape(1, C))

    return outT.T.reshape(1, C, N)


# speedup vs baseline: 1.2022x; 1.2022x over previous
import functools

import numpy as np
import jax
import jax.numpy as jnp
from jax import lax
from jax.experimental import pallas as pl
from jax.experimental.pallas import tpu as pltpu
from jax.experimental.pallas import tpu_sc as plsc

C = 128
N = 10000
K = 32

NC = 2
NS = 16
NW = NC * NS
L = 16
CV = C // L

TN = 4
NCHUNK = N // TN
BASE_CNT = NCHUNK // NW
EXTRA = NCHUNK - BASE_CNT * NW
MAXCNT = BASE_CNT + 1
IDXPAD = MAXCNT * TN * K


def _prep_body(f_ref, w_ref, b_ref, o_ref):
    acc = lax.dot(f_ref[...], w_ref[...],
                  preferred_element_type=jnp.float32)
    h = jnp.maximum(acc + b_ref[...], 0.0)
    bits = lax.bitcast_convert_type(h, jnp.int32)
    rne = (bits + 0x7FFF + ((bits >> 16) & 1)) >> 16
    parts = []
    for g in range(4):
        lo = rne[:, 32 * g:32 * g + 16] & 0xFFFF
        hi = rne[:, 32 * g + 16:32 * g + 32] << 16
        parts.append(lo | hi)
    o_ref[...] = jnp.concatenate(parts, axis=1)


def _tail_body(g_ref, f_ref, w1_ref, b1_ref, w2_ref, b2_ref, o_ref):
    t = lax.dot(g_ref[...], w1_ref[...],
                preferred_element_type=jnp.float32)
    t = jnp.maximum(t + b1_ref[...], 0.0)
    o = lax.dot(t, w2_ref[...],
                preferred_element_type=jnp.float32)
    o_ref[...] = jnp.maximum(o + b2_ref[...] + f_ref[...], 0.0)


NBUF = 3
NOBUF = 2


def _sc_core_body(h_hbm, idx_hbm, dlp_hbm, out_hbm,
                  idx_v, dlp_v, rows_v, out_v,
                  dlp_sem, gat_sem, out_sem):
    wid = lax.axis_index("s") * NC + lax.axis_index("c")
    cnt = BASE_CNT + jnp.where(wid < EXTRA, 1, 0)
    start = wid * BASE_CNT + jnp.minimum(wid, EXTRA)

    pltpu.sync_copy(idx_hbm.at[pl.ds(start * TN * K, IDXPAD)], idx_v)

    def dlp_copy(slot, i):
        return pltpu.make_async_copy(
            dlp_hbm.at[pl.ds((start + i) * TN * K * C, TN * K * C)],
            dlp_v.at[slot], dlp_sem.at[slot])

    def gat_copy(slot, i):
        return pltpu.make_async_copy(
            h_hbm.at[idx_v.at[pl.ds(i * TN * K, TN * K)]],
            rows_v.at[slot], gat_sem.at[slot])

    def out_copy(slot, i):
        return pltpu.make_async_copy(
            out_v.at[slot],
            out_hbm.at[pl.ds((start + i) * TN * C, TN * C)], out_sem.at[slot])

    for pre in range(NBUF - 1):
        @pl.when(pre < cnt)
        def _():
            dlp_copy(pre, pre).start()
            gat_copy(pre, pre).start()

    def chunk_body(i, _):
        s = lax.rem(i, NBUF)
        so = lax.rem(i, NOBUF)

        @pl.when(i + NBUF - 1 < cnt)
        def _():
            dlp_copy(lax.rem(i + NBUF - 1, NBUF), i + NBUF - 1).start()
            gat_copy(lax.rem(i + NBUF - 1, NBUF), i + NBUF - 1).start()

        dlp_copy(s, i).wait()
        gat_copy(s, i).wait()

        @pl.when(i >= NOBUF)
        def _():
            out_copy(so, i - NOBUF).wait()

        for nn in range(TN):
            base = nn * K * C

            def row_terms(k):
                terms = []
                for g in range(CV // 2):
                    vi = rows_v[s, nn * K + k, pl.ds(g * L, L)]
                    vb = plsc.bitcast(vi, jnp.bfloat16)
                    a, b = plsc.unpack(vb, format=plsc.PackFormat.INTERLEAVED)
                    terms += [a, b]
                return terms

            def k_body(k, acc):
                off = base + k * C
                t = row_terms(k)
                return tuple(
                    jnp.maximum(acc[j],
                                dlp_v[s, pl.ds(off + j * L, L)] + t[j])
                    for j in range(CV))

            t0 = row_terms(0)
            acc0 = tuple(
                dlp_v[s, pl.ds(base + j * L, L)] + t0[j]
                for j in range(CV))
            acc = lax.fori_loop(1, K, k_body, acc0, unroll=4)
            for j in range(CV):
                out_v[so, pl.ds(nn * C + j * L, L)] = acc[j]

        out_copy(so, i).start()
        return 0

    lax.fori_loop(0, cnt, chunk_body, 0)

    for tail in range(NOBUF):
        @pl.when(cnt > tail)
        def _():
            i = cnt - 1 - tail
            out_copy(lax.rem(i, NOBUF), i).wait()


@functools.cache
def _sc_core():
    return pl.kernel(
        _sc_core_body,
        out_type=jax.ShapeDtypeStruct((N * C,), jnp.float32),
        mesh=plsc.VectorSubcoreMesh(core_axis_name="c", subcore_axis_name="s",
                                    num_cores=NC, num_subcores=NS),
        compiler_params=pltpu.CompilerParams(needs_layout_passes=False,
                                             use_tc_tiling_on_sc=False),
        scratch_types=[
            pltpu.VMEM((IDXPAD,), jnp.int32),
            pltpu.VMEM((NBUF, TN * K * C), jnp.float32),
            pltpu.VMEM((NBUF, TN * K, C // 2), jnp.int32),
            pltpu.VMEM((NOBUF, TN * C), jnp.float32),
            pltpu.SemaphoreType.DMA((NBUF,)),
            pltpu.SemaphoreType.DMA((NBUF,)),
            pltpu.SemaphoreType.DMA((NOBUF,)),
        ],
    )


def kernel(p, f, dlp, idx, W1, b1, Wp1, bp1, Wp2, bp2):
    del p
    fT = f.reshape(C, N).T
    dlp_flat = jnp.transpose(dlp.reshape(C, N, K), (1, 2, 0)).reshape(-1)
    idx_flat = idx.reshape(N * K).astype(jnp.int32)
    idx_flat = jnp.pad(idx_flat, (0, NW * MAXCNT * TN * K - N * K))

    TB = 2000
    grid = (N // TB,)
    hT = pl.pallas_call(
        _prep_body,
        grid=grid,
        in_specs=[
            pl.BlockSpec((TB, C), lambda i: (i, 0)),
            pl.BlockSpec((C, C), lambda i: (0, 0)),
            pl.BlockSpec((1, C), lambda i: (0, 0)),
        ],
        out_specs=pl.BlockSpec((TB, C // 2), lambda i: (i, 0)),
        out_shape=jax.ShapeDtypeStruct((N, C // 2), jnp.int32),
    )(fT, W1.T, b1.reshape(1, C))

    g0 = _sc_core()(hT, idx_flat, dlp_flat).reshape(N, C)

    outT = pl.pallas_call(
        _tail_body,
        grid=grid,
        in_specs=[
            pl.BlockSpec((TB, C), lambda i: (i, 0)),
            pl.BlockSpec((TB, C), lambda i: (i, 0)),
            pl.BlockSpec((C, C), lambda i: (0, 0)),
            pl.BlockSpec((1, C), lambda i: (0, 0)),
            pl.BlockSpec((C, C), lambda i: (0, 0)),
            pl.BlockSpec((1, C), lambda i: (0, 0)),
        ],
        out_specs=pl.BlockSpec((TB, C), lambda i: (i, 0)),
        out_shape=jax.ShapeDtypeStruct((N, C), jnp.float32),
    )(g0, fT, Wp1.T, bp1.reshape(1, C), Wp2.T, bp2.reshape(1, C))

    return outT.T.reshape(1, C, N)
